# baseline (speedup 1.0000x reference)
import functools

import jax
import jax.numpy as jnp
from jax import lax
from jax.experimental import pallas as pl
from jax.experimental.pallas import tpu as pltpu
from jax.experimental.pallas import tpu_sc as plsc

_NUM_FIELDS = 26
_VOCAB = 100000
_EMBED = 16
_HIDDEN = 256
_BATCH = 16384

_NC = 2
_NS = 16
_NW = _NC * _NS

_CB = 128
_CH = _CB * _NUM_FIELDS
_LANES = 16


def _gather_sc(indices, flat_table):
    F, B = indices.shape
    b_per_w = B // _NW
    n_chunks = b_per_w // _CB

    mesh = plsc.VectorSubcoreMesh(core_axis_name="c", subcore_axis_name="s")

    @functools.partial(
        pl.kernel,
        mesh=mesh,
        compiler_params=pltpu.CompilerParams(use_tc_tiling_on_sc=False),
        out_type=jax.ShapeDtypeStruct((B, F, _EMBED), jnp.float32),
        scratch_types=[
            pltpu.VMEM((_NUM_FIELDS, _CB), jnp.int32),
            pltpu.VMEM((_CH,), jnp.int32),
            pltpu.VMEM((_CH, _EMBED), jnp.float32),
            pltpu.SemaphoreType.DMA,
        ],
    )
    def k(idx_hbm, tab_hbm, out_hbm, blk_v, ids_v, rows_v, sem):
        wid = lax.axis_index("s") * _NC + lax.axis_index("c")
        b0 = wid * b_per_w

        def chunk(j, _):
            bj = b0 + j * _CB
            pltpu.sync_copy(idx_hbm.at[:, pl.ds(bj, _CB)], blk_v)
            for f in range(F):
                for t in range(_CB // _LANES):
                    s = pl.ds(t * _LANES, _LANES)
                    ids_v[pl.ds(f * _CB + t * _LANES, _LANES)] = (
                        blk_v[f, s] + f * _VOCAB
                    )
            pltpu.async_copy(tab_hbm.at[ids_v], rows_v, sem).wait()
            copies = [
                pltpu.async_copy(
                    rows_v.at[pl.ds(f * _CB, _CB)],
                    out_hbm.at[pl.ds(bj, _CB), f],
                    sem,
                )
                for f in range(F)
            ]
            for c in copies:
                c.wait()
            return 0

        lax.fori_loop(0, n_chunks, chunk, 0)

    return k(indices, flat_table)


def _project_tc(feats, w, b):
    B, K = feats.shape
    H = w.shape[1]
    blk = 2048

    def mm(f_ref, w_ref, b_ref, o_ref):
        o_ref[...] = (
            jnp.dot(f_ref[...], w_ref[...], preferred_element_type=jnp.float32)
            + b_ref[...]
        )

    return pl.pallas_call(
        mm,
        grid=(B // blk,),
        in_specs=[
            pl.BlockSpec((blk, K), lambda i: (i, 0)),
            pl.BlockSpec((K, H), lambda i: (0, 0)),
            pl.BlockSpec((1, H), lambda i: (0, 0)),
        ],
        out_specs=pl.BlockSpec((blk, H), lambda i: (i, 0)),
        out_shape=jax.ShapeDtypeStruct((B, H), jnp.float32),
    )(feats, w, b.reshape(1, H))


def kernel(indices, tables, proj_w, proj_b):
    F, B = indices.shape
    V, E = tables.shape[1], tables.shape[2]
    flat_table = tables.reshape(F * V, E)
    feats = _gather_sc(indices, flat_table).reshape(B, F * E)
    return _project_tc(feats, proj_w, proj_b)
